# Initial kernel scaffold; baseline (speedup 1.0000x reference)
import jax
import jax.numpy as jnp
from jax.experimental import pallas as pl


def kernel(pts, nn_idx, We1, be1, We2, be2, Wg, bg, Wd, bd):
    raise NotImplementedError("write your pallas kernel here")


# speedup vs baseline: 3.8750x; 3.8750x over previous
import functools

import jax
import jax.numpy as jnp
from jax.experimental import pallas as pl
from jax.experimental.pallas import tpu as pltpu

ANUM = 2
N_ETYPES = 8
E_HID = 64
C = 128
K = 16


def _dense_body(knn3_ref, ptsT_ref, We1T_ref, be1_ref, We2T_ref, be2_ref,
                WgT_ref, bg_ref, WdT_ref, bd_ref, out_ref):
    nb = knn3_ref.shape[0]
    knn3 = knn3_ref[...]
    center = ptsT_ref[...]
    efeat3 = knn3 - center[:, None, :]
    efeat = efeat3.reshape(nb * K, C)
    h = jnp.dot(efeat, We1T_ref[...], preferred_element_type=jnp.float32,
                precision=jax.lax.Precision.HIGHEST)
    h = jax.nn.relu(h + be1_ref[...][None, :])
    logits = jnp.dot(h, We2T_ref[...], preferred_element_type=jnp.float32,
                     precision=jax.lax.Precision.HIGHEST)
    logits = logits + be2_ref[...][None, :]
    m = jnp.max(logits, axis=-1, keepdims=True)
    e = jnp.exp(logits - m)
    etype = e / jnp.sum(e, axis=-1, keepdims=True)
    etype3 = etype.reshape(nb, K, N_ETYPES)
    aggs = []
    for t in range(N_ETYPES):
        w = etype3[:, :, t][:, :, None]
        aggs.append(jnp.sum(w * knn3, axis=1))
    aggf = jnp.concatenate(aggs, axis=-1) * (1.0 / K)
    for a in range(ANUM):
        msg = jnp.dot(aggf, WgT_ref[a], preferred_element_type=jnp.float32,
                      precision=jax.lax.Precision.HIGHEST)
        x = jax.nn.relu(center + msg + bg_ref[a][None, :])
        y = jnp.dot(x, WdT_ref[a], preferred_element_type=jnp.float32,
                    precision=jax.lax.Precision.HIGHEST)
        out_ref[a] = y + bd_ref[a][None, :]


def _dense_call(knn3, ptsT, We1T, be1, We2T, be2, WgT, bg, WdT, bd, *, nb):
    n = ptsT.shape[0]
    hid2 = WdT.shape[-1]
    grid = (n // nb,)
    rep = lambda *_: (0,) * 3
    out = pl.pallas_call(
        _dense_body,
        grid=grid,
        in_specs=[
            pl.BlockSpec((nb, K, C), lambda i: (i, 0, 0)),
            pl.BlockSpec((nb, C), lambda i: (i, 0)),
            pl.BlockSpec((C, E_HID), lambda i: (0, 0)),
            pl.BlockSpec((E_HID,), lambda i: (0,)),
            pl.BlockSpec((E_HID, N_ETYPES), lambda i: (0, 0)),
            pl.BlockSpec((N_ETYPES,), lambda i: (0,)),
            pl.BlockSpec((ANUM, N_ETYPES * C, C), rep),
            pl.BlockSpec((ANUM, C), lambda i: (0, 0)),
            pl.BlockSpec((ANUM, C, hid2), rep),
            pl.BlockSpec((ANUM, hid2), lambda i: (0, 0)),
        ],
        out_specs=pl.BlockSpec((ANUM, nb, hid2), lambda i: (0, i, 0)),
        out_shape=jax.ShapeDtypeStruct((ANUM, n, hid2), jnp.float32),
    )(knn3, ptsT, We1T, be1, We2T, be2, WgT, bg, WdT, bd)
    return out


def kernel(pts, nn_idx, We1, be1, We2, be2, Wg, bg, Wd, bd):
    B, c, N = pts.shape
    k = nn_idx.shape[-1]
    hid2 = Wd.shape[1]
    ptsT = pts[0].T
    flat_idx = nn_idx[0].reshape(N * k)
    knn3 = jnp.take(ptsT, flat_idx, axis=0).reshape(N, k, c)
    out = _dense_call(
        knn3, ptsT,
        We1.T, be1, We2.T, be2,
        jnp.transpose(Wg, (0, 2, 1)), bg,
        jnp.transpose(Wd, (0, 2, 1)), bd,
        nb=200,
    )
    y = jnp.transpose(out, (0, 2, 1))
    return y[None, :, :, :, None]


# speedup vs baseline: 6.6025x; 1.7039x over previous
import functools

import jax
import jax.numpy as jnp
from jax.experimental import pallas as pl
from jax.experimental.pallas import tpu as pltpu

ANUM = 2
N_ETYPES = 8
E_HID = 64
C = 128
K = 16

_PREC = jax.lax.Precision.DEFAULT


def _dot(a, b):
    return jnp.dot(a, b, preferred_element_type=jnp.float32, precision=_PREC)


def _dense_body(pts_ref, knn_ref, We1_ref, be1_ref, We2_ref, be2_ref,
                Wg_ref, bg_ref, Wd_ref, bd_ref, out_ref):
    nb = pts_ref.shape[1]
    ctr = pts_ref[...]
    We1 = We1_ref[...]
    be1 = be1_ref[...]
    We2 = We2_ref[...]
    be2 = be2_ref[...]
    knnTs = []
    logits = []
    for k in range(K):
        pkT = knn_ref[k].T
        knnTs.append(pkT)
        h = jnp.maximum(_dot(We1, pkT - ctr) + be1, 0.0)
        logits.append(_dot(We2, h) + be2)
    L3 = jnp.concatenate(logits, axis=0).reshape(K, N_ETYPES, nb)
    m = jnp.max(L3, axis=1, keepdims=True)
    p = jnp.exp(L3 - m)
    s = jnp.sum(p, axis=1, keepdims=True)
    w3 = p / s
    aggs = []
    for t in range(N_ETYPES):
        acc = w3[0, t][None, :] * knnTs[0]
        for k in range(1, K):
            acc = acc + w3[k, t][None, :] * knnTs[k]
        aggs.append(acc)
    aggf = jnp.concatenate(aggs, axis=0) * (1.0 / K)
    for a in range(ANUM):
        msg = _dot(Wg_ref[a], aggf) + bg_ref[a]
        x = jnp.maximum(ctr + msg, 0.0)
        out_ref[a] = _dot(Wd_ref[a], x) + bd_ref[a]


def _dense_call(pts2, knn_planes, We1, be1c, We2, be2c, Wg, bgc, Wd, bdc, *, nb):
    n = pts2.shape[1]
    hid2 = Wd.shape[1]
    grid = (n // nb,)
    return pl.pallas_call(
        _dense_body,
        grid=grid,
        in_specs=[
            pl.BlockSpec((C, nb), lambda i: (0, i)),
            pl.BlockSpec((K, nb, C), lambda i: (0, i, 0)),
            pl.BlockSpec((E_HID, C), lambda i: (0, 0)),
            pl.BlockSpec((E_HID, 1), lambda i: (0, 0)),
            pl.BlockSpec((N_ETYPES, E_HID), lambda i: (0, 0)),
            pl.BlockSpec((N_ETYPES, 1), lambda i: (0, 0)),
            pl.BlockSpec((ANUM, C, N_ETYPES * C), lambda i: (0, 0, 0)),
            pl.BlockSpec((ANUM, C, 1), lambda i: (0, 0, 0)),
            pl.BlockSpec((ANUM, hid2, C), lambda i: (0, 0, 0)),
            pl.BlockSpec((ANUM, hid2, 1), lambda i: (0, 0, 0)),
        ],
        out_specs=pl.BlockSpec((ANUM, hid2, nb), lambda i: (0, 0, i)),
        out_shape=jax.ShapeDtypeStruct((ANUM, hid2, n), jnp.float32),
    )(pts2, knn_planes, We1, be1c, We2, be2c, Wg, bgc, Wd, bdc)


def kernel(pts, nn_idx, We1, be1, We2, be2, Wg, bg, Wd, bd):
    B, c, N = pts.shape
    k = nn_idx.shape[-1]
    NP = 10240
    pts2 = jnp.pad(pts[0], ((0, 0), (0, NP - N)))
    ptsT = pts[0].T
    idxT = jnp.pad(nn_idx[0].T, ((0, 0), (0, NP - N)))
    knn_planes = jnp.take(ptsT, idxT.reshape(-1), axis=0).reshape(k, NP, c)
    out = _dense_call(
        pts2, knn_planes,
        We1, be1[:, None], We2, be2[:, None],
        Wg, bg[:, :, None], Wd, bd[:, :, None],
        nb=1024,
    )
    return out[None, :, :, :N, None]


# speedup vs baseline: 14.0647x; 2.1302x over previous
import functools

import jax
import jax.numpy as jnp
from jax import lax
from jax.experimental import pallas as pl
from jax.experimental.pallas import tpu as pltpu
from jax.experimental.pallas import tpu_sc as plsc

ANUM = 2
N_ETYPES = 8
E_HID = 64
C = 128
K = 16

_PREC = jax.lax.Precision.DEFAULT


def _dot(a, b):
    return jnp.dot(a, b, preferred_element_type=jnp.float32, precision=_PREC)


def _dense_body(pts_ref, knn_ref, We1_ref, be1_ref, We2_ref, be2_ref,
                Wg_ref, bg_ref, Wd_ref, bd_ref, out_ref):
    nb = pts_ref.shape[1]
    ctr = pts_ref[...]
    We1 = We1_ref[...]
    be1 = be1_ref[...]
    We2 = We2_ref[...]
    be2 = be2_ref[...]
    knnTs = []
    logits = []
    for k in range(K):
        pkT = knn_ref[k].T
        knnTs.append(pkT)
        h = jnp.maximum(_dot(We1, pkT - ctr) + be1, 0.0)
        logits.append(_dot(We2, h) + be2)
    L3 = jnp.concatenate(logits, axis=0).reshape(K, N_ETYPES, nb)
    m = jnp.max(L3, axis=1, keepdims=True)
    p = jnp.exp(L3 - m)
    s = jnp.sum(p, axis=1, keepdims=True)
    w3 = p / s
    aggs = []
    for t in range(N_ETYPES):
        acc = w3[0, t][None, :] * knnTs[0]
        for k in range(1, K):
            acc = acc + w3[k, t][None, :] * knnTs[k]
        aggs.append(acc)
    aggf = jnp.concatenate(aggs, axis=0) * (1.0 / K)
    for a in range(ANUM):
        msg = _dot(Wg_ref[a], aggf) + bg_ref[a]
        x = jnp.maximum(ctr + msg, 0.0)
        out_ref[a] = _dot(Wd_ref[a], x) + bd_ref[a]


def _dense_call(pts2, knn_planes, We1, be1c, We2, be2c, Wg, bgc, Wd, bdc, *, nb):
    n = pts2.shape[1]
    hid2 = Wd.shape[1]
    grid = (n // nb,)
    return pl.pallas_call(
        _dense_body,
        grid=grid,
        in_specs=[
            pl.BlockSpec((C, nb), lambda i: (0, i)),
            pl.BlockSpec((K, nb, C), lambda i: (0, i, 0)),
            pl.BlockSpec((E_HID, C), lambda i: (0, 0)),
            pl.BlockSpec((E_HID, 1), lambda i: (0, 0)),
            pl.BlockSpec((N_ETYPES, E_HID), lambda i: (0, 0)),
            pl.BlockSpec((N_ETYPES, 1), lambda i: (0, 0)),
            pl.BlockSpec((ANUM, C, N_ETYPES * C), lambda i: (0, 0, 0)),
            pl.BlockSpec((ANUM, C, 1), lambda i: (0, 0, 0)),
            pl.BlockSpec((ANUM, hid2, C), lambda i: (0, 0, 0)),
            pl.BlockSpec((ANUM, hid2, 1), lambda i: (0, 0, 0)),
        ],
        out_specs=pl.BlockSpec((ANUM, hid2, nb), lambda i: (0, 0, i)),
        out_shape=jax.ShapeDtypeStruct((ANUM, hid2, n), jnp.float32),
    )(pts2, knn_planes, We1, be1c, We2, be2c, Wg, bgc, Wd, bdc)


_NP = 10240
_CHUNK = 128
_NCHUNK = _NP // 2 // _CHUNK


def _sc_gather(ptsT, idx4):
    mesh = plsc.VectorSubcoreMesh(core_axis_name="c", subcore_axis_name="s")

    @functools.partial(
        pl.kernel,
        mesh=mesh,
        out_type=jax.ShapeDtypeStruct((K, 2, _NCHUNK, _CHUNK, C), jnp.float32),
        scratch_types=[
            pltpu.VMEM((_NCHUNK, _CHUNK), jnp.int32),
            pltpu.VMEM((_CHUNK, C), jnp.float32),
            pltpu.VMEM((_CHUNK, C), jnp.float32),
            pltpu.SemaphoreType.DMA,
            pltpu.SemaphoreType.DMA,
        ],
    )
    def gkern(ptsT_hbm, idx_hbm, out_hbm, idx_v, rows0, rows1, sem0, sem1):
        wid = lax.axis_index("s") * 2 + lax.axis_index("c")
        plane = wid // 2
        half = wid % 2
        pltpu.sync_copy(idx_hbm.at[plane, half], idx_v)
        rows = (rows0, rows1)
        sems = (sem0, sem1)

        def step(j, carry):
            for b in range(2):
                ch = 2 * j + b
                g = pltpu.async_copy(ptsT_hbm.at[idx_v.at[ch]], rows[b], sems[b])
                g.wait()
                pltpu.sync_copy(rows[b], out_hbm.at[plane, half, ch])
            return carry

        lax.fori_loop(0, _NCHUNK // 2, step, 0)

    return gkern(ptsT, idx4)


def kernel(pts, nn_idx, We1, be1, We2, be2, Wg, bg, Wd, bd):
    B, c, N = pts.shape
    k = nn_idx.shape[-1]
    NP = _NP
    pts2 = jnp.pad(pts[0], ((0, 0), (0, NP - N)))
    ptsT = pts[0].T
    idxT = jnp.pad(nn_idx[0].T, ((0, 0), (0, NP - N)))
    idx4 = idxT.reshape(k, 2, _NCHUNK, _CHUNK)
    knn_planes = _sc_gather(ptsT, idx4).reshape(k, NP, c)
    out = _dense_call(
        pts2, knn_planes,
        We1, be1[:, None], We2, be2[:, None],
        Wg, bg[:, :, None], Wd, bd[:, :, None],
        nb=1024,
    )
    return out[None, :, :, :N, None]


# speedup vs baseline: 14.8100x; 1.0530x over previous
import functools

import jax
import jax.numpy as jnp
from jax import lax
from jax.experimental import pallas as pl
from jax.experimental.pallas import tpu as pltpu
from jax.experimental.pallas import tpu_sc as plsc

ANUM = 2
N_ETYPES = 8
E_HID = 64
C = 128
K = 16

_PREC = jax.lax.Precision.DEFAULT


def _dot(a, b):
    return jnp.dot(a, b, preferred_element_type=jnp.float32, precision=_PREC)


def _dense_body(pts_ref, knn_ref, We1_ref, be1_ref, We2_ref, be2_ref,
                Wg_ref, bg_ref, Wd_ref, bd_ref, out_ref):
    nb = pts_ref.shape[1]
    ctr = pts_ref[...]
    We1 = We1_ref[...]
    be1 = be1_ref[...]
    We2 = We2_ref[...]
    be2 = be2_ref[...]
    knnTs = []
    logits = []
    for k in range(K):
        pkT = knn_ref[k].T
        knnTs.append(pkT)
        h = jnp.maximum(_dot(We1, pkT - ctr) + be1, 0.0)
        logits.append(_dot(We2, h) + be2)
    L3 = jnp.concatenate(logits, axis=0).reshape(K, N_ETYPES, nb)
    m = jnp.max(L3, axis=1, keepdims=True)
    p = jnp.exp(L3 - m)
    s = jnp.sum(p, axis=1, keepdims=True)
    w3 = p / s
    aggs = []
    for t in range(N_ETYPES):
        acc = w3[0, t][None, :] * knnTs[0]
        for k in range(1, K):
            acc = acc + w3[k, t][None, :] * knnTs[k]
        aggs.append(acc)
    aggf = jnp.concatenate(aggs, axis=0) * (1.0 / K)
    for a in range(ANUM):
        msg = _dot(Wg_ref[a], aggf) + bg_ref[a]
        x = jnp.maximum(ctr + msg, 0.0)
        out_ref[a] = _dot(Wd_ref[a], x) + bd_ref[a]


def _dense_call(pts2, knn_planes, We1, be1c, We2, be2c, Wg, bgc, Wd, bdc, *, nb):
    n = pts2.shape[1]
    hid2 = Wd.shape[1]
    grid = (n // nb,)
    return pl.pallas_call(
        _dense_body,
        grid=grid,
        in_specs=[
            pl.BlockSpec((C, nb), lambda i: (0, i)),
            pl.BlockSpec((K, nb, C), lambda i: (0, i, 0)),
            pl.BlockSpec((E_HID, C), lambda i: (0, 0)),
            pl.BlockSpec((E_HID, 1), lambda i: (0, 0)),
            pl.BlockSpec((N_ETYPES, E_HID), lambda i: (0, 0)),
            pl.BlockSpec((N_ETYPES, 1), lambda i: (0, 0)),
            pl.BlockSpec((ANUM, C, N_ETYPES * C), lambda i: (0, 0, 0)),
            pl.BlockSpec((ANUM, C, 1), lambda i: (0, 0, 0)),
            pl.BlockSpec((ANUM, hid2, C), lambda i: (0, 0, 0)),
            pl.BlockSpec((ANUM, hid2, 1), lambda i: (0, 0, 0)),
        ],
        out_specs=pl.BlockSpec((ANUM, hid2, nb), lambda i: (0, 0, i)),
        out_shape=jax.ShapeDtypeStruct((ANUM, hid2, n), jnp.float32),
    )(pts2, knn_planes, We1, be1c, We2, be2c, Wg, bgc, Wd, bdc)


_NP = 10240
_CHUNK = 128
_NCHUNK = _NP // 2 // _CHUNK


def _sc_gather(ptsT, idx4):
    mesh = plsc.VectorSubcoreMesh(core_axis_name="c", subcore_axis_name="s")

    NBUF = 4

    @functools.partial(
        pl.kernel,
        mesh=mesh,
        out_type=jax.ShapeDtypeStruct((K, 2, _NCHUNK, _CHUNK, C), jnp.float32),
        scratch_types=(
            [pltpu.VMEM((_NCHUNK, _CHUNK), jnp.int32)]
            + [pltpu.VMEM((_CHUNK, C), jnp.float32)] * NBUF
            + [pltpu.SemaphoreType.DMA] * (2 * NBUF)
        ),
    )
    def gkern(ptsT_hbm, idx_hbm, out_hbm, idx_v, *bufs):
        rows = bufs[:NBUF]
        gsem = bufs[NBUF:2 * NBUF]
        wsem = bufs[2 * NBUF:]
        wid = lax.axis_index("s") * 2 + lax.axis_index("c")
        plane = wid // 2
        half = wid % 2
        pltpu.sync_copy(idx_hbm.at[plane, half], idx_v)

        def gather(b, ch):
            pltpu.async_copy(ptsT_hbm.at[idx_v.at[ch]], rows[b], gsem[b])

        def gather_wait(b, ch):
            pltpu.make_async_copy(ptsT_hbm.at[idx_v.at[ch]], rows[b], gsem[b]).wait()

        def write(b, ch):
            pltpu.async_copy(rows[b], out_hbm.at[plane, half, ch], wsem[b])

        def write_wait(b, ch):
            pltpu.make_async_copy(rows[b], out_hbm.at[plane, half, ch], wsem[b]).wait()

        for b in range(NBUF):
            gather(b, b)

        def step(j, carry):
            for b in range(NBUF):
                ch = NBUF * j + b
                gather_wait(b, ch)
                write(b, ch)
            for b in range(NBUF):
                ch = NBUF * j + b
                write_wait(b, ch)

                @pl.when(j < _NCHUNK // NBUF - 1)
                def _():
                    gather(b, ch + NBUF)
            return carry

        lax.fori_loop(0, _NCHUNK // NBUF, step, 0)

    return gkern(ptsT, idx4)


def kernel(pts, nn_idx, We1, be1, We2, be2, Wg, bg, Wd, bd):
    B, c, N = pts.shape
    k = nn_idx.shape[-1]
    NP = _NP
    pts2 = jnp.pad(pts[0], ((0, 0), (0, NP - N)))
    ptsT = pts[0].T
    idxT = jnp.pad(nn_idx[0].T, ((0, 0), (0, NP - N)))
    idx4 = idxT.reshape(k, 2, _NCHUNK, _CHUNK)
    knn_planes = _sc_gather(ptsT, idx4).reshape(k, NP, c)
    out = _dense_call(
        pts2, knn_planes,
        We1, be1[:, None], We2, be2[:, None],
        Wg, bg[:, :, None], Wd, bd[:, :, None],
        nb=1024,
    )
    return out[None, :, :, :N, None]
